# baseline (speedup 1.0000x reference)
import functools

import jax
import jax.numpy as jnp
from jax import lax
from jax.experimental import pallas as pl
from jax.experimental.pallas import tpu as pltpu
from jax.experimental.pallas import tpu_sc as plsc

_NVEC1 = 1000
_NC = 2
_NS = 16
_NW = _NC * _NS
_LANES = 16
_CHUNK = 128


def kernel(xs, table):
    B = xs.shape[0]
    b_per_w = B // _NW
    n_chunks = b_per_w // _CHUNK
    rows_per_w = 2 * n_chunks
    per_chunk = _CHUNK // _LANES

    mesh = plsc.VectorSubcoreMesh(core_axis_name="c", subcore_axis_name="s")

    @functools.partial(
        pl.kernel,
        mesh=mesh,
        compiler_params=pltpu.CompilerParams(use_tc_tiling_on_sc=False),
        out_type=jax.ShapeDtypeStruct((B,), jnp.float32),
        scratch_types=[
            pltpu.VMEM((rows_per_w, _CHUNK), jnp.int32),
            pltpu.VMEM((n_chunks, _CHUNK), jnp.int32),
            pltpu.VMEM((b_per_w,), jnp.float32),
            pltpu.SemaphoreType.DMA,
            pltpu.SemaphoreType.DMA,
        ],
    )
    def _k(xsv_hbm, table_hbm, out_hbm, xs_v, idx_v, vals_v, in_sem, gat_sem):
        wid = lax.axis_index("s") * _NC + lax.axis_index("c")
        base = wid * b_per_w

        pltpu.async_copy(
            xsv_hbm.at[pl.ds(wid * rows_per_w, rows_per_w), :], xs_v, in_sem
        ).wait()

        copies = []
        for j in range(n_chunks):
            for i in range(per_chunk):
                x0 = xs_v[2 * j, pl.ds(i * _LANES, _LANES)]
                x1 = xs_v[2 * j + 1, pl.ds(i * _LANES, _LANES)]
                idx_v[j, pl.ds(i * _LANES, _LANES)] = x0 * _NVEC1 + x1
            copies.append(
                pltpu.async_copy(
                    table_hbm.at[idx_v.at[j]],
                    vals_v.at[pl.ds(j * _CHUNK, _CHUNK)],
                    gat_sem,
                )
            )
        out_copies = []
        for j, c in enumerate(copies):
            c.wait()
            out_copies.append(
                pltpu.async_copy(
                    vals_v.at[pl.ds(j * _CHUNK, _CHUNK)],
                    out_hbm.at[pl.ds(base + j * _CHUNK, _CHUNK)],
                    in_sem,
                )
            )
        for c in out_copies:
            c.wait()

    xs_view = xs.reshape(B // _CHUNK, _CHUNK, 2).transpose(0, 2, 1)
    xs_view = xs_view.reshape(2 * (B // _CHUNK), _CHUNK)
    return _k(xs_view, table.reshape(-1))


# speedup vs baseline: 2.2577x; 2.2577x over previous
import functools

import jax
import jax.numpy as jnp
from jax import lax
from jax.experimental import pallas as pl
from jax.experimental.pallas import tpu as pltpu
from jax.experimental.pallas import tpu_sc as plsc

_NVEC1 = 1000
_NC = 2
_NS = 16
_NW = _NC * _NS
_LANES = 16
_CHUNK = 128


def kernel(xs, table):
    B = xs.shape[0]
    b_per_w = B // _NW
    n_chunks = b_per_w // _CHUNK
    rows_per_w = 2 * n_chunks
    per_chunk = _CHUNK // _LANES

    mesh = plsc.VectorSubcoreMesh(core_axis_name="c", subcore_axis_name="s")

    @functools.partial(
        pl.kernel,
        mesh=mesh,
        compiler_params=pltpu.CompilerParams(skip_device_barrier=True),
        out_type=jax.ShapeDtypeStruct((B,), jnp.float32),
        scratch_types=[
            pltpu.VMEM((rows_per_w, _CHUNK), jnp.int32),
            pltpu.VMEM((n_chunks, _CHUNK), jnp.int32),
            pltpu.VMEM((b_per_w,), jnp.float32),
            pltpu.SemaphoreType.DMA,
            pltpu.SemaphoreType.DMA,
        ],
    )
    def _k(xsv_hbm, table_hbm, out_hbm, xs_v, idx_v, vals_v, in_sem, gat_sem):
        wid = lax.axis_index("s") * _NC + lax.axis_index("c")
        base = wid * b_per_w

        pltpu.async_copy(
            xsv_hbm.at[pl.ds(wid * rows_per_w, rows_per_w), :], xs_v, in_sem
        ).wait()

        copies = []
        for j in range(n_chunks):
            for i in range(per_chunk):
                x0 = xs_v[2 * j, pl.ds(i * _LANES, _LANES)]
                x1 = xs_v[2 * j + 1, pl.ds(i * _LANES, _LANES)]
                idx_v[j, pl.ds(i * _LANES, _LANES)] = x0 * _NVEC1 + x1
            copies.append(
                pltpu.async_copy(
                    table_hbm.at[idx_v.at[j]],
                    vals_v.at[pl.ds(j * _CHUNK, _CHUNK)],
                    gat_sem,
                )
            )
        out_copies = []
        for j, c in enumerate(copies):
            c.wait()
            out_copies.append(
                pltpu.async_copy(
                    vals_v.at[pl.ds(j * _CHUNK, _CHUNK)],
                    out_hbm.at[pl.ds(base + j * _CHUNK, _CHUNK)],
                    in_sem,
                )
            )
        for c in out_copies:
            c.wait()

    xs_view = xs.reshape(B // _CHUNK, _CHUNK, 2).transpose(0, 2, 1)
    xs_view = xs_view.reshape(2 * (B // _CHUNK), _CHUNK)
    pad = (-table.shape[0]) % 1024
    tp = jnp.pad(table, ((0, pad), (0, 0)))
    return _k(xs_view, tp.reshape(-1))


# speedup vs baseline: 2.2663x; 1.0038x over previous
import functools

import jax
import jax.numpy as jnp
from jax import lax
from jax.experimental import pallas as pl
from jax.experimental.pallas import tpu as pltpu
from jax.experimental.pallas import tpu_sc as plsc

_NVEC1 = 1000
_NC = 2
_NS = 16
_NW = _NC * _NS
_LANES = 16
_CHUNK = 128


def kernel(xs, table):
    B = xs.shape[0]
    b_per_w = B // _NW
    n_chunks = b_per_w // _CHUNK
    rows_per_w = 2 * n_chunks
    per_chunk = _CHUNK // _LANES

    mesh = plsc.VectorSubcoreMesh(core_axis_name="c", subcore_axis_name="s")

    @functools.partial(
        pl.kernel,
        mesh=mesh,
        out_type=jax.ShapeDtypeStruct((B,), jnp.float32),
        scratch_types=[
            pltpu.VMEM((rows_per_w, _CHUNK), jnp.int32),
            pltpu.VMEM((n_chunks, _CHUNK), jnp.int32),
            pltpu.VMEM((b_per_w,), jnp.float32),
            pltpu.SemaphoreType.DMA,
            pltpu.SemaphoreType.DMA,
        ],
    )
    def _k(xsv_hbm, table_hbm, out_hbm, xs_v, idx_v, vals_v, in_sem, gat_sem):
        wid = lax.axis_index("s") * _NC + lax.axis_index("c")
        base = wid * b_per_w

        pltpu.async_copy(
            xsv_hbm.at[pl.ds(wid * rows_per_w, rows_per_w), :], xs_v, in_sem
        ).wait()

        copies = []
        for j in range(n_chunks):
            for i in range(per_chunk):
                x0 = xs_v[2 * j, pl.ds(i * _LANES, _LANES)]
                x1 = xs_v[2 * j + 1, pl.ds(i * _LANES, _LANES)]
                idx_v[j, pl.ds(i * _LANES, _LANES)] = x0 * _NVEC1 + x1
            copies.append(
                pltpu.async_copy(
                    table_hbm.at[idx_v.at[j]],
                    vals_v.at[pl.ds(j * _CHUNK, _CHUNK)],
                    gat_sem,
                )
            )
        out_copies = []
        for j, c in enumerate(copies):
            c.wait()
            out_copies.append(
                pltpu.async_copy(
                    vals_v.at[pl.ds(j * _CHUNK, _CHUNK)],
                    out_hbm.at[pl.ds(base + j * _CHUNK, _CHUNK)],
                    in_sem,
                )
            )
        for c in out_copies:
            c.wait()

    xs_view = xs.reshape(B // _CHUNK, _CHUNK, 2).transpose(0, 2, 1)
    xs_view = xs_view.reshape(2 * (B // _CHUNK), _CHUNK)
    pad = (-table.shape[0]) % 1024
    tp = lax.dynamic_update_slice(
        jnp.zeros((table.shape[0] + pad, 1), jnp.float32), table, (0, 0)
    )
    return _k(xs_view, tp.reshape(-1))


# speedup vs baseline: 2.2710x; 1.0021x over previous
import functools

import jax
import jax.numpy as jnp
from jax import lax
from jax.experimental import pallas as pl
from jax.experimental.pallas import tpu as pltpu
from jax.experimental.pallas import tpu_sc as plsc

_NVEC1 = 1000
_NC = 2
_NS = 16
_NW = _NC * _NS
_LANES = 16
_CHUNK = 128


def kernel(xs, table):
    B = xs.shape[0]
    b_per_w = B // _NW
    n_chunks = b_per_w // _CHUNK
    rows_per_w = 2 * n_chunks
    per_chunk = _CHUNK // _LANES

    mesh = plsc.VectorSubcoreMesh(core_axis_name="c", subcore_axis_name="s")

    @functools.partial(
        pl.kernel,
        mesh=mesh,
        out_type=jax.ShapeDtypeStruct((B,), jnp.float32),
        scratch_types=[
            pltpu.VMEM((rows_per_w, _CHUNK), jnp.int32),
            pltpu.VMEM((n_chunks, _CHUNK), jnp.int32),
            pltpu.VMEM((b_per_w,), jnp.float32),
            pltpu.SemaphoreType.DMA,
            pltpu.SemaphoreType.DMA,
        ],
    )
    def _k(xsv_hbm, table_hbm, out_hbm, xs_v, idx_v, vals_v, in_sem, gat_sem):
        wid = lax.axis_index("s") * _NC + lax.axis_index("c")
        base = wid * b_per_w

        in_copies = [
            pltpu.async_copy(
                xsv_hbm.at[pl.ds(wid * rows_per_w + 2 * j, 2), :],
                xs_v.at[pl.ds(2 * j, 2), :],
                in_sem,
            )
            for j in range(n_chunks)
        ]

        copies = []
        for j in range(n_chunks):
            in_copies[j].wait()
            for i in range(per_chunk):
                x0 = xs_v[2 * j, pl.ds(i * _LANES, _LANES)]
                x1 = xs_v[2 * j + 1, pl.ds(i * _LANES, _LANES)]
                idx_v[j, pl.ds(i * _LANES, _LANES)] = x0 * _NVEC1 + x1
            copies.append(
                pltpu.async_copy(
                    table_hbm.at[idx_v.at[j]],
                    vals_v.at[pl.ds(j * _CHUNK, _CHUNK)],
                    gat_sem,
                )
            )
        out_copies = []
        for j, c in enumerate(copies):
            c.wait()
            out_copies.append(
                pltpu.async_copy(
                    vals_v.at[pl.ds(j * _CHUNK, _CHUNK)],
                    out_hbm.at[pl.ds(base + j * _CHUNK, _CHUNK)],
                    in_sem,
                )
            )
        for c in out_copies:
            c.wait()

    xs_view = xs.reshape(B // _CHUNK, _CHUNK, 2).transpose(0, 2, 1)
    xs_view = xs_view.reshape(2 * (B // _CHUNK), _CHUNK)
    pad = (-table.shape[0]) % 1024
    tp = lax.dynamic_update_slice(
        jnp.zeros((table.shape[0] + pad, 1), jnp.float32), table, (0, 0)
    )
    return _k(xs_view, tp.reshape(-1))


# speedup vs baseline: 2.2713x; 1.0001x over previous
import functools

import jax
import jax.numpy as jnp
from jax import lax
from jax.experimental import pallas as pl
from jax.experimental.pallas import tpu as pltpu
from jax.experimental.pallas import tpu_sc as plsc

_NVEC1 = 1000
_NC = 2
_NS = 16
_NW = _NC * _NS
_LANES = 16
_CHUNK = 128


def kernel(xs, table):
    B = xs.shape[0]
    b_per_w = B // _NW
    n_chunks = b_per_w // _CHUNK
    rows_per_w = 2 * n_chunks
    per_chunk = _CHUNK // _LANES

    mesh = plsc.VectorSubcoreMesh(core_axis_name="c", subcore_axis_name="s")

    @functools.partial(
        pl.kernel,
        mesh=mesh,
        out_type=jax.ShapeDtypeStruct((B,), jnp.float32),
        scratch_types=[
            pltpu.VMEM((rows_per_w, _CHUNK), jnp.int32),
            pltpu.VMEM((n_chunks, _CHUNK), jnp.int32),
            pltpu.VMEM((b_per_w,), jnp.float32),
            pltpu.SemaphoreType.DMA,
            pltpu.SemaphoreType.DMA,
        ],
    )
    def _k(xsv_hbm, table_hbm, out_hbm, xs_v, idx_v, vals_v, in_sem, gat_sem):
        wid = lax.axis_index("s") * _NC + lax.axis_index("c")
        base = wid * b_per_w

        in_copies = [
            pltpu.async_copy(
                xsv_hbm.at[pl.ds(wid * rows_per_w + 2 * j, 2), :],
                xs_v.at[pl.ds(2 * j, 2), :],
                in_sem,
            )
            for j in range(n_chunks)
        ]

        copies = []
        for j in range(n_chunks):
            in_copies[j].wait()
            for i in range(per_chunk):
                x0 = xs_v[2 * j, pl.ds(i * _LANES, _LANES)]
                x1 = xs_v[2 * j + 1, pl.ds(i * _LANES, _LANES)]
                idx_v[j, pl.ds(i * _LANES, _LANES)] = x0 * _NVEC1 + x1
            copies.append(
                pltpu.async_copy(
                    table_hbm.at[idx_v.at[j]],
                    vals_v.at[pl.ds(j * _CHUNK, _CHUNK)],
                    gat_sem,
                )
            )
        out_copies = []
        for j, c in enumerate(copies):
            c.wait()
            out_copies.append(
                pltpu.async_copy(
                    vals_v.at[pl.ds(j * _CHUNK, _CHUNK)],
                    out_hbm.at[pl.ds(base + j * _CHUNK, _CHUNK)],
                    in_sem,
                )
            )
        for c in out_copies:
            c.wait()

    xs_view = xs.reshape(B // _CHUNK, _CHUNK, 2).transpose(0, 2, 1)
    xs_view = xs_view.reshape(2 * (B // _CHUNK), _CHUNK)
    pad = (-table.shape[0]) % 1024
    tp = jnp.pad(table, ((0, pad), (0, 0)))
    return _k(xs_view, tp.reshape(-1))
